# baseline (speedup 1.0000x reference)
import functools

import jax
import jax.numpy as jnp
from jax import lax
from jax.experimental import pallas as pl
from jax.experimental.pallas import tpu as pltpu
from jax.experimental.pallas import tpu_sc as plsc

VOCAB = 50265
HIDDEN = 768
BATCH = 64
SEQ = 512
EPS = 1e-12

NUM_WORKERS = 32
TOKENS = BATCH * SEQ
TOK_PER_W = TOKENS // NUM_WORKERS
CHUNK = 32
NCHUNK = TOK_PER_W // CHUNK

ROWS_BLK = 4


NBUF = 4


def _sc_gather(table, ids):
    mesh = plsc.VectorSubcoreMesh(core_axis_name="c", subcore_axis_name="s")

    @functools.partial(
        pl.kernel,
        out_type=jax.ShapeDtypeStruct((TOKENS, HIDDEN), jnp.float32),
        mesh=mesh,
        scratch_types=[
            pltpu.VMEM((TOK_PER_W,), jnp.int32),
        ]
        + [pltpu.VMEM((CHUNK, HIDDEN), jnp.float32) for _ in range(NBUF)]
        + [pltpu.SemaphoreType.DMA for _ in range(2 * NBUF)],
    )
    def gather_kernel(table_hbm, idx_hbm, out_hbm, idx_v, *rest):
        bufs = rest[:NBUF]
        gsem = rest[NBUF : 2 * NBUF]
        ssem = rest[2 * NBUF :]
        wid = lax.axis_index("s") * 2 + lax.axis_index("c")
        base = wid * TOK_PER_W
        pltpu.sync_copy(idx_hbm.at[pl.ds(base, TOK_PER_W)], idx_v)

        def gather_desc(cur, b):
            return pltpu.make_async_copy(
                table_hbm.at[idx_v.at[pl.ds(cur * CHUNK, CHUNK)]], bufs[b], gsem[b]
            )

        def store_desc(cur, b):
            return pltpu.make_async_copy(
                bufs[b], out_hbm.at[pl.ds(base + cur * CHUNK, CHUNK)], ssem[b]
            )

        gather_desc(0, 0).start()
        gather_desc(1, 1).start()

        @pl.loop(0, NCHUNK, step=NBUF)
        def _(i):
            for b in range(NBUF):
                cur = i + b
                gather_desc(cur, b).wait()
                store_desc(cur, b).start()
                nxt = cur + 2

                @pl.when(nxt < NCHUNK)
                def _():
                    bn = (b + 2) % NBUF

                    @pl.when(nxt >= NBUF)
                    def _():
                        store_desc(nxt - NBUF, bn).wait()

                    gather_desc(nxt, bn).start()

        for c in range(NCHUNK - NBUF, NCHUNK):
            store_desc(c, c % NBUF).wait()

    return gather_kernel(table, ids)


def _ln_body(x_ref, pos_ref, w_ref, b_ref, o_ref):
    inv = 1.0 / HIDDEN
    for r in range(ROWS_BLK):
        x = x_ref[r] + pos_ref[...]
        u = jnp.sum(x, axis=1, keepdims=True) * inv
        s = jnp.sum(x * x, axis=1, keepdims=True) * inv - u * u
        rstd = lax.rsqrt(s + EPS)
        y = (x - u) * (rstd * w_ref[...]) + b_ref[...]
        o_ref[r] = y.T


def _ln_transpose(gathered, pos, w, b):
    return pl.pallas_call(
        _ln_body,
        grid=(BATCH // ROWS_BLK,),
        in_specs=[
            pl.BlockSpec((ROWS_BLK, SEQ, HIDDEN), lambda i: (i, 0, 0)),
            pl.BlockSpec((SEQ, HIDDEN), lambda i: (0, 0)),
            pl.BlockSpec((1, HIDDEN), lambda i: (0, 0)),
            pl.BlockSpec((1, HIDDEN), lambda i: (0, 0)),
        ],
        out_specs=pl.BlockSpec((ROWS_BLK, HIDDEN, SEQ), lambda i: (i, 0, 0)),
        out_shape=jax.ShapeDtypeStruct((BATCH, HIDDEN, SEQ), jnp.float32),
        compiler_params=pltpu.CompilerParams(
            dimension_semantics=("arbitrary",),
        ),
    )(gathered, pos, w, b)


@jax.jit
def kernel(input_ids, word_embeddings, position_embeddings, ln_weight, ln_bias):
    ids = input_ids.reshape(-1).astype(jnp.int32)
    gathered = _sc_gather(word_embeddings, ids)
    gathered = gathered.reshape(BATCH, SEQ, HIDDEN)
    pos = position_embeddings[:SEQ]
    w = ln_weight.reshape(1, HIDDEN)
    b = ln_bias.reshape(1, HIDDEN)
    return _ln_transpose(gathered, pos, w, b)
